# baseline (speedup 1.0000x reference)
import jax
import jax.numpy as jnp
from jax.experimental import pallas as pl
from jax.experimental.pallas import tpu as pltpu


def _probe_kernel(adj_ref, s_ref, out_ref):
    out_ref[...] = jnp.dot(adj_ref[...], s_ref[...],
                           precision=jax.lax.Precision.HIGHEST,
                           preferred_element_type=jnp.float32)


def kernel(input, adj, W, b):
    B, N, F_in = input.shape
    F_out = W.shape[1]
    BM = 200
    out = pl.pallas_call(
        _probe_kernel,
        grid=(N // BM,),
        in_specs=[pl.BlockSpec((BM, N), lambda i: (i, 0)),
                  pl.BlockSpec((N, 128), lambda i: (0, 0))],
        out_specs=pl.BlockSpec((BM, 128), lambda i: (i, 0)),
        out_shape=jax.ShapeDtypeStruct((N, 128), jnp.float32),
        compiler_params=pltpu.CompilerParams(
            dimension_semantics=("arbitrary",)),
    )(adj, input.reshape(N, F_in))
    return out.reshape(1, N, 128)


# speedup vs baseline: 2.7213x; 2.7213x over previous
import jax
import jax.numpy as jnp
from jax.experimental import pallas as pl
from jax.experimental.pallas import tpu as pltpu

_RING = 4
_LEAD = 3


def _gcn_kernel(adj_hbm, x_ref, w_ref, b_ref, out_ref, s_ref, abuf, sems):
    i = pl.program_id(0)
    n_i = pl.num_programs(0)
    bm = out_ref.shape[0]

    @pl.when(i == 0)
    def _():
        s_ref[...] = jnp.dot(x_ref[...], w_ref[...],
                             preferred_element_type=jnp.float32)
        for j in range(_LEAD):
            @pl.when(j < n_i)
            def _():
                pltpu.make_async_copy(
                    adj_hbm.at[pl.ds(j * bm, bm), :],
                    abuf.at[j], sems.at[j]).start()

    nxt = i + _LEAD

    @pl.when(nxt < n_i)
    def _():
        pltpu.make_async_copy(
            adj_hbm.at[pl.ds(nxt * bm, bm), :],
            abuf.at[nxt % _RING], sems.at[nxt % _RING]).start()

    slot = i % _RING
    pltpu.make_async_copy(
        adj_hbm.at[pl.ds(i * bm, bm), :],
        abuf.at[slot], sems.at[slot]).wait()
    out_ref[...] = (
        jnp.dot(abuf[slot], s_ref[...], preferred_element_type=jnp.float32)
        + b_ref[...]
    )


def _gcn_single(x, adj, W, b2):
    N, F_in = x.shape
    F_out = W.shape[1]

    BM = min(200, N)
    return pl.pallas_call(
        _gcn_kernel,
        grid=(N // BM,),
        in_specs=[
            pl.BlockSpec(memory_space=pltpu.MemorySpace.HBM),
            pl.BlockSpec((N, F_in), lambda i: (0, 0)),
            pl.BlockSpec((F_in, F_out), lambda i: (0, 0)),
            pl.BlockSpec((1, F_out), lambda i: (0, 0)),
        ],
        out_specs=pl.BlockSpec((BM, F_out), lambda i: (i, 0)),
        out_shape=jax.ShapeDtypeStruct((N, F_out), jnp.float32),
        scratch_shapes=[
            pltpu.VMEM((N, F_out), jnp.float32),
            pltpu.VMEM((_RING, BM, N), jnp.float32),
            pltpu.SemaphoreType.DMA((_RING,)),
        ],
        compiler_params=pltpu.CompilerParams(
            dimension_semantics=("arbitrary",)),
    )(adj, x, W, b2)


def kernel(input, adj, W, b):
    B, N, F_in = input.shape
    F_out = W.shape[1]
    b2 = b.reshape(1, F_out)
    outs = [_gcn_single(input[i], adj, W, b2) for i in range(B)]
    return jnp.stack(outs, axis=0)


# speedup vs baseline: 2.7493x; 1.0103x over previous
import jax
import jax.numpy as jnp
from jax.experimental import pallas as pl
from jax.experimental.pallas import tpu as pltpu


def _gcn_kernel(adj_ref, x_ref, w_ref, b_ref, out_ref, s_ref):
    @pl.when(pl.program_id(0) == 0)
    def _():
        s_ref[...] = jnp.dot(x_ref[...], w_ref[...],
                             preferred_element_type=jnp.float32
                             ).astype(jnp.bfloat16)

    out_ref[...] = (
        jax.lax.dot_general(
            adj_ref[...], s_ref[...], (((1,), (0,)), ((), ())),
            preferred_element_type=jnp.float32)
        + b_ref[...]
    )


def _gcn_single(x, adj, W, b2):
    N, F_in = x.shape
    F_out = W.shape[1]

    BM = min(200, N)
    return pl.pallas_call(
        _gcn_kernel,
        grid=(N // BM,),
        in_specs=[
            pl.BlockSpec((BM, N), lambda i: (i, 0)),
            pl.BlockSpec((N, F_in), lambda i: (0, 0)),
            pl.BlockSpec((F_in, F_out), lambda i: (0, 0)),
            pl.BlockSpec((1, F_out), lambda i: (0, 0)),
        ],
        out_specs=pl.BlockSpec((BM, F_out), lambda i: (i, 0)),
        out_shape=jax.ShapeDtypeStruct((N, F_out), jnp.float32),
        scratch_shapes=[pltpu.VMEM((N, F_out), jnp.bfloat16)],
        compiler_params=pltpu.CompilerParams(
            dimension_semantics=("arbitrary",)),
    )(adj, x, W, b2)


def kernel(input, adj, W, b):
    B, N, F_in = input.shape
    F_out = W.shape[1]
    b2 = b.reshape(1, F_out)
    outs = [_gcn_single(input[i], adj, W, b2) for i in range(B)]
    return jnp.stack(outs, axis=0)


# speedup vs baseline: 2.7540x; 1.0017x over previous
import jax
import jax.numpy as jnp
from jax.experimental import pallas as pl
from jax.experimental.pallas import tpu as pltpu


def _gcn_kernel(adj_ref, x_ref, w_ref, b_ref, out_ref, s_ref):
    @pl.when(pl.program_id(0) == 0)
    def _():
        s_ref[...] = jnp.dot(x_ref[...], w_ref[...],
                             preferred_element_type=jnp.float32)

    out_ref[...] = (
        jnp.dot(adj_ref[...], s_ref[...], preferred_element_type=jnp.float32)
        + b_ref[...]
    )


def _gcn_single(x, adj, W, b2):
    N, F_in = x.shape
    F_out = W.shape[1]

    BM = min(200, N)
    return pl.pallas_call(
        _gcn_kernel,
        grid=(N // BM,),
        in_specs=[
            pl.BlockSpec((BM, N), lambda i: (i, 0)),
            pl.BlockSpec((N, F_in), lambda i: (0, 0)),
            pl.BlockSpec((F_in, F_out), lambda i: (0, 0)),
            pl.BlockSpec((1, F_out), lambda i: (0, 0)),
        ],
        out_specs=pl.BlockSpec((BM, F_out), lambda i: (i, 0)),
        out_shape=jax.ShapeDtypeStruct((N, F_out), jnp.float32),
        scratch_shapes=[pltpu.VMEM((N, F_out), jnp.float32)],
        compiler_params=pltpu.CompilerParams(
            dimension_semantics=("arbitrary",)),
    )(adj, x, W, b2)


def kernel(input, adj, W, b):
    B, N, F_in = input.shape
    F_out = W.shape[1]
    b2 = b.reshape(1, F_out)
    outs = [_gcn_single(input[i], adj, W, b2) for i in range(B)]
    return jnp.stack(outs, axis=0)
